# baseline (speedup 1.0000x reference)
import functools

import jax
import jax.numpy as jnp
from jax import lax
from jax.experimental import pallas as pl
from jax.experimental.pallas import tpu as pltpu
from jax.experimental.pallas import tpu_sc as plsc

SEQ = 16384
DIM = 64
VOCAB = 1000000
NUM_CORES = 2
NUM_SUBCORES = 16
NUM_TILES = NUM_CORES * NUM_SUBCORES
PER_SUBCORE = SEQ // NUM_SUBCORES
CHUNK = 128
NCHUNK = PER_SUBCORE // CHUNK
L = 16

BV = 8192
NB_HALF = 62
HALF = NB_HALF * BV
H_PER_TILE = HALF // NUM_SUBCORES

T = 29
TC_NB = T + 1
TC_GRID = NUM_CORES * TC_NB
WIN = 512
CC = WIN // L


def _sc_multiplicity(X):
    mesh = plsc.VectorSubcoreMesh(core_axis_name="c", subcore_axis_name="s")

    @functools.partial(
        pl.kernel,
        mesh=mesh,
        out_type=jax.ShapeDtypeStruct((NUM_CORES, HALF), jnp.float32),
        scratch_types=[
            pltpu.VMEM((NCHUNK, CHUNK), jnp.int32),
            pltpu.VMEM((CHUNK,), jnp.float32),
            pltpu.VMEM((H_PER_TILE,), jnp.float32),
            pltpu.VMEM_SHARED((HALF + L,), jnp.float32),
        ],
    )
    def sc_kernel(x_hbm, m_hbm, idx_v, ones_v, zeros_v, m_sh):
        cid = lax.axis_index("c")
        sid = lax.axis_index("s")
        base = sid * PER_SUBCORE
        lo = cid * HALF
        dump = HALF + (sid % L)

        for j in range(NCHUNK):
            pltpu.sync_copy(
                x_hbm.at[pl.ds(base + j * CHUNK, CHUNK)], idx_v.at[j]
            )
        for j in range(NCHUNK):
            for k in range(CHUNK // L):
                v = idx_v[j, pl.ds(k * L, L)] - lo
                ok = (v >= 0) & (v < HALF)
                idx_v[j, pl.ds(k * L, L)] = jnp.where(ok, v, dump)

        for k in range(CHUNK // L):
            ones_v[pl.ds(k * L, L)] = jnp.full((L,), 1.0, jnp.float32)

        def zbody(i, _):
            zeros_v[pl.ds(i * L, L)] = jnp.zeros((L,), jnp.float32)
            return 0

        lax.fori_loop(0, H_PER_TILE // L, zbody, 0, unroll=8)

        tslice = pl.ds(sid * H_PER_TILE, H_PER_TILE)
        pltpu.sync_copy(zeros_v, m_sh.at[tslice])
        plsc.subcore_barrier()

        for j in range(NCHUNK):
            pltpu.sync_copy(ones_v, m_sh.at[idx_v.at[j]], add=True)
        plsc.subcore_barrier()

        pltpu.sync_copy(m_sh.at[tslice], m_hbm.at[cid, tslice])

    return sc_kernel(X)


def _sc_stream(ET, m):
    mesh = plsc.VectorSubcoreMesh(core_axis_name="c", subcore_axis_name="s")

    @functools.partial(
        pl.kernel,
        mesh=mesh,
        out_type=jax.ShapeDtypeStruct((NUM_TILES, DIM, L), jnp.float32),
        scratch_types=[
            pltpu.VMEM((2, 8, WIN), jnp.float32),
            pltpu.VMEM((2, WIN), jnp.float32),
            pltpu.VMEM((DIM, L), jnp.float32),
            pltpu.SemaphoreType.DMA((2,)),
            pltpu.SemaphoreType.DMA((2,)),
        ],
    )
    def sc_kernel(et_hbm, m_hbm, out_hbm, ebuf, mbuf, acc_v, esem, msem):
        cid = lax.axis_index("c")
        sid = lax.axis_index("s")
        wid = cid * NUM_SUBCORES + sid
        nblk = jnp.where(cid == 0, NB_HALF - 1 - T, NB_HALF - 2 - T)
        nchunks = nblk * 8

        def zinit(r, _):
            acc_v[r] = jnp.zeros((L,), jnp.float32)
            return 0

        lax.fori_loop(0, DIM, zinit, 0)

        def col0_of(blk):
            return (cid * NB_HALF + T + blk) * BV + sid * WIN

        def moff_of(blk):
            return (T + blk) * BV + sid * WIN

        def fire_e(ci, p):
            blk, e8 = ci // 8, ci % 8
            return pltpu.async_copy(
                et_hbm.at[pl.ds(e8 * 8, 8), pl.ds(col0_of(blk), WIN)],
                ebuf.at[p],
                esem.at[p],
            )

        def fire_m(blk, p):
            return pltpu.async_copy(
                m_hbm.at[cid, pl.ds(moff_of(blk), WIN)],
                mbuf.at[p],
                msem.at[p],
            )

        def drain_e(p):
            pltpu.make_async_copy(
                et_hbm.at[pl.ds(0, 8), pl.ds(0, WIN)], ebuf.at[p], esem.at[p]
            ).wait()

        def drain_m(p):
            pltpu.make_async_copy(
                m_hbm.at[0, pl.ds(0, WIN)], mbuf.at[p], msem.at[p]
            ).wait()

        fire_e(0, 0)
        fire_m(0, 0)

        def body(ci, _):
            p = lax.rem(ci, 2)
            blk = ci // 8

            @pl.when(ci + 1 < nchunks)
            def _():
                fire_e(ci + 1, 1 - p)

            mp = lax.rem(blk, 2)

            @pl.when((ci % 8 == 0) & (blk + 1 < nblk))
            def _():
                fire_m(blk + 1, 1 - mp)

            drain_e(p)

            @pl.when(ci % 8 == 0)
            def _():
                drain_m(mp)

            e8 = ci % 8

            def inner(cc, accs):
                m16 = mbuf[mp, pl.ds(cc * L, L)]
                out = []
                for r in range(8):
                    x = jnp.maximum(ebuf[p, r, pl.ds(cc * L, L)], 0.0)
                    out.append(accs[r] + x * m16)
                return tuple(out)

            z = jnp.zeros((L,), jnp.float32)
            accs = lax.fori_loop(0, CC, inner, (z,) * 8, unroll=2)
            for r in range(8):
                acc_v[e8 * 8 + r] = acc_v[e8 * 8 + r] + accs[r]
            return 0

        lax.fori_loop(0, nchunks, body, 0)
        pltpu.sync_copy(acc_v, out_hbm.at[wid])

    return sc_kernel(ET, m)


def _tc_stream(ET, m):
    def tc_kernel(e_ref, m_ref, o_ref, acc_ref):
        i = pl.program_id(0)
        r = i // TC_NB
        j = i % TC_NB
        sel = jnp.where(j < T, j, NB_HALF - 1 - r)
        base = (r * NB_HALF + sel) * BV
        cols = jax.lax.broadcasted_iota(jnp.int32, (1, BV), 1) + base
        e = jnp.where(cols < VOCAB, e_ref[...], 0.0)
        mm = jnp.where(r == 0, m_ref[0:1, :], m_ref[1:2, :])
        contrib = jnp.sum(jnp.maximum(e, 0.0) * mm, axis=1, keepdims=True)

        @pl.when(i == 0)
        def _():
            acc_ref[...] = contrib

        @pl.when(i > 0)
        def _():
            acc_ref[...] = acc_ref[...] + contrib

        @pl.when(i == TC_GRID - 1)
        def _():
            o_ref[...] = acc_ref[...]

    def eidx(i):
        r = i // TC_NB
        j = i % TC_NB
        sel = jnp.where(j < T, j, NB_HALF - 1 - r)
        return (0, r * NB_HALF + sel)

    def midx(i):
        r = i // TC_NB
        j = i % TC_NB
        sel = jnp.where(j < T, j, NB_HALF - 1 - r)
        return (0, sel)

    return pl.pallas_call(
        tc_kernel,
        grid=(TC_GRID,),
        in_specs=[
            pl.BlockSpec((DIM, BV), eidx),
            pl.BlockSpec((NUM_CORES, BV), midx),
        ],
        out_specs=pl.BlockSpec((DIM, 1), lambda i: (0, 0)),
        out_shape=jax.ShapeDtypeStruct((DIM, 1), jnp.float32),
        scratch_shapes=[pltpu.VMEM((DIM, 1), jnp.float32)],
    )(ET, m)


def _tc_head(hidden_tc, partials, W, b2):
    def tc_kernel(h_ref, p_ref, w_ref, b_ref, o_ref):
        ps = jnp.sum(p_ref[...], axis=0)
        ones = jnp.ones((L, 1), jnp.float32)
        h_sc = lax.dot_general(
            ps, ones, (((1,), (0,)), ((), ())),
            preferred_element_type=jnp.float32,
        )
        hidden = h_ref[...] + h_sc
        logits = lax.dot_general(
            w_ref[...], hidden, (((1,), (0,)), ((), ())),
            preferred_element_type=jnp.float32,
        ) + b_ref[...]
        mx = jnp.max(logits, axis=0, keepdims=True)
        ex = jnp.exp(logits - mx)
        o_ref[...] = ex / jnp.sum(ex, axis=0, keepdims=True)

    return pl.pallas_call(
        tc_kernel,
        out_shape=jax.ShapeDtypeStruct((2, 1), jnp.float32),
    )(hidden_tc, partials, W, b2)


def kernel(X, E, W, b):
    X = X.astype(jnp.int32)
    ET = E.T
    m = _sc_multiplicity(X)
    hidden_tc = _tc_stream(ET, m)
    partials = _sc_stream(ET, m)
    out = _tc_head(hidden_tc, partials, W, b.reshape(2, 1))
    return out.reshape(2)


# speedup vs baseline: 1.2768x; 1.2768x over previous
import functools

import jax
import jax.numpy as jnp
from jax import lax
from jax.experimental import pallas as pl
from jax.experimental.pallas import tpu as pltpu
from jax.experimental.pallas import tpu_sc as plsc

SEQ = 16384
DIM = 64
VOCAB = 1000000
NUM_CORES = 2
NUM_SUBCORES = 16
NUM_TILES = NUM_CORES * NUM_SUBCORES
PER_SUBCORE = SEQ // NUM_SUBCORES
CHUNK = 128
NCHUNK = PER_SUBCORE // CHUNK
L = 16

BV = 8192
NB_HALF = 62
HALF = NB_HALF * BV
H_PER_TILE = HALF // NUM_SUBCORES

T = 41
TC_NB = T + 1
TC_GRID = NUM_CORES * TC_NB
WIN = 512
CC = WIN // L


def _sc_multiplicity(X):
    mesh = plsc.VectorSubcoreMesh(core_axis_name="c", subcore_axis_name="s")

    @functools.partial(
        pl.kernel,
        mesh=mesh,
        out_type=jax.ShapeDtypeStruct((NUM_CORES, HALF), jnp.float32),
        scratch_types=[
            pltpu.VMEM((NCHUNK, CHUNK), jnp.int32),
            pltpu.VMEM((CHUNK,), jnp.float32),
            pltpu.VMEM((H_PER_TILE,), jnp.float32),
            pltpu.VMEM_SHARED((HALF + L,), jnp.float32),
        ],
    )
    def sc_kernel(x_hbm, m_hbm, idx_v, ones_v, zeros_v, m_sh):
        cid = lax.axis_index("c")
        sid = lax.axis_index("s")
        base = sid * PER_SUBCORE
        lo = cid * HALF
        dump = HALF + (sid % L)

        for j in range(NCHUNK):
            pltpu.sync_copy(
                x_hbm.at[pl.ds(base + j * CHUNK, CHUNK)], idx_v.at[j]
            )
        for j in range(NCHUNK):
            for k in range(CHUNK // L):
                v = idx_v[j, pl.ds(k * L, L)] - lo
                ok = (v >= 0) & (v < HALF)
                idx_v[j, pl.ds(k * L, L)] = jnp.where(ok, v, dump)

        for k in range(CHUNK // L):
            ones_v[pl.ds(k * L, L)] = jnp.full((L,), 1.0, jnp.float32)

        def zbody(i, _):
            zeros_v[pl.ds(i * L, L)] = jnp.zeros((L,), jnp.float32)
            return 0

        lax.fori_loop(0, H_PER_TILE // L, zbody, 0, unroll=8)

        tslice = pl.ds(sid * H_PER_TILE, H_PER_TILE)
        pltpu.sync_copy(zeros_v, m_sh.at[tslice])
        plsc.subcore_barrier()

        for j in range(NCHUNK):
            pltpu.sync_copy(ones_v, m_sh.at[idx_v.at[j]], add=True)
        plsc.subcore_barrier()

        pltpu.sync_copy(m_sh.at[tslice], m_hbm.at[cid, tslice])

    return sc_kernel(X)


def _sc_stream(ET, m):
    mesh = plsc.VectorSubcoreMesh(core_axis_name="c", subcore_axis_name="s")

    @functools.partial(
        pl.kernel,
        mesh=mesh,
        out_type=jax.ShapeDtypeStruct((NUM_TILES, DIM, L), jnp.float32),
        scratch_types=[
            pltpu.VMEM((2, 8, WIN), jnp.float32),
            pltpu.VMEM((2, WIN), jnp.float32),
            pltpu.VMEM((DIM, L), jnp.float32),
            pltpu.SemaphoreType.DMA((2,)),
            pltpu.SemaphoreType.DMA((2,)),
        ],
    )
    def sc_kernel(et_hbm, m_hbm, out_hbm, ebuf, mbuf, acc_v, esem, msem):
        cid = lax.axis_index("c")
        sid = lax.axis_index("s")
        wid = cid * NUM_SUBCORES + sid
        nblk = jnp.where(cid == 0, NB_HALF - 1 - T, NB_HALF - 2 - T)
        nchunks = nblk * 8

        def zinit(r, _):
            acc_v[r] = jnp.zeros((L,), jnp.float32)
            return 0

        lax.fori_loop(0, DIM, zinit, 0)

        def col0_of(blk):
            return (cid * NB_HALF + T + blk) * BV + sid * WIN

        def moff_of(blk):
            return (T + blk) * BV + sid * WIN

        def fire_e(ci, p):
            blk, e8 = ci // 8, ci % 8
            return pltpu.async_copy(
                et_hbm.at[pl.ds(e8 * 8, 8), pl.ds(col0_of(blk), WIN)],
                ebuf.at[p],
                esem.at[p],
            )

        def fire_m(blk, p):
            return pltpu.async_copy(
                m_hbm.at[cid, pl.ds(moff_of(blk), WIN)],
                mbuf.at[p],
                msem.at[p],
            )

        def drain_e(p):
            pltpu.make_async_copy(
                et_hbm.at[pl.ds(0, 8), pl.ds(0, WIN)], ebuf.at[p], esem.at[p]
            ).wait()

        def drain_m(p):
            pltpu.make_async_copy(
                m_hbm.at[0, pl.ds(0, WIN)], mbuf.at[p], msem.at[p]
            ).wait()

        fire_e(0, 0)
        fire_m(0, 0)

        def body(ci, _):
            p = lax.rem(ci, 2)
            blk = ci // 8

            @pl.when(ci + 1 < nchunks)
            def _():
                fire_e(ci + 1, 1 - p)

            mp = lax.rem(blk, 2)

            @pl.when((ci % 8 == 0) & (blk + 1 < nblk))
            def _():
                fire_m(blk + 1, 1 - mp)

            drain_e(p)

            @pl.when(ci % 8 == 0)
            def _():
                drain_m(mp)

            e8 = ci % 8

            def inner(cc, accs):
                m16 = mbuf[mp, pl.ds(cc * L, L)]
                out = []
                for r in range(8):
                    x = jnp.maximum(ebuf[p, r, pl.ds(cc * L, L)], 0.0)
                    out.append(accs[r] + x * m16)
                return tuple(out)

            z = jnp.zeros((L,), jnp.float32)
            accs = lax.fori_loop(0, CC, inner, (z,) * 8, unroll=4)
            for r in range(8):
                acc_v[e8 * 8 + r] = acc_v[e8 * 8 + r] + accs[r]
            return 0

        lax.fori_loop(0, nchunks, body, 0)
        pltpu.sync_copy(acc_v, out_hbm.at[wid])

    return sc_kernel(ET, m)


def _tc_stream(ET, m):
    def tc_kernel(e_ref, m_ref, o_ref, acc_ref):
        i = pl.program_id(0)
        r = i // TC_NB
        j = i % TC_NB
        sel = jnp.where(j < T, j, NB_HALF - 1 - r)
        base = (r * NB_HALF + sel) * BV
        cols = jax.lax.broadcasted_iota(jnp.int32, (1, BV), 1) + base
        e = jnp.where(cols < VOCAB, e_ref[...], 0.0)
        mm = jnp.where(r == 0, m_ref[0:1, :], m_ref[1:2, :])
        contrib = jnp.sum(jnp.maximum(e, 0.0) * mm, axis=1, keepdims=True)

        @pl.when(i == 0)
        def _():
            acc_ref[...] = contrib

        @pl.when(i > 0)
        def _():
            acc_ref[...] = acc_ref[...] + contrib

        @pl.when(i == TC_GRID - 1)
        def _():
            o_ref[...] = acc_ref[...]

    def eidx(i):
        r = i // TC_NB
        j = i % TC_NB
        sel = jnp.where(j < T, j, NB_HALF - 1 - r)
        return (0, r * NB_HALF + sel)

    def midx(i):
        r = i // TC_NB
        j = i % TC_NB
        sel = jnp.where(j < T, j, NB_HALF - 1 - r)
        return (0, sel)

    return pl.pallas_call(
        tc_kernel,
        grid=(TC_GRID,),
        in_specs=[
            pl.BlockSpec((DIM, BV), eidx),
            pl.BlockSpec((NUM_CORES, BV), midx),
        ],
        out_specs=pl.BlockSpec((DIM, 1), lambda i: (0, 0)),
        out_shape=jax.ShapeDtypeStruct((DIM, 1), jnp.float32),
        scratch_shapes=[pltpu.VMEM((DIM, 1), jnp.float32)],
    )(ET, m)


def _tc_head(hidden_tc, partials, W, b2):
    def tc_kernel(h_ref, p_ref, w_ref, b_ref, o_ref):
        ps = jnp.sum(p_ref[...], axis=0)
        ones = jnp.ones((L, 1), jnp.float32)
        h_sc = lax.dot_general(
            ps, ones, (((1,), (0,)), ((), ())),
            preferred_element_type=jnp.float32,
        )
        hidden = h_ref[...] + h_sc
        logits = lax.dot_general(
            w_ref[...], hidden, (((1,), (0,)), ((), ())),
            preferred_element_type=jnp.float32,
        ) + b_ref[...]
        mx = jnp.max(logits, axis=0, keepdims=True)
        ex = jnp.exp(logits - mx)
        o_ref[...] = ex / jnp.sum(ex, axis=0, keepdims=True)

    return pl.pallas_call(
        tc_kernel,
        out_shape=jax.ShapeDtypeStruct((2, 1), jnp.float32),
    )(hidden_tc, partials, W, b2)


def kernel(X, E, W, b):
    X = X.astype(jnp.int32)
    ET = E.T
    m = _sc_multiplicity(X)
    hidden_tc = _tc_stream(ET, m)
    partials = _sc_stream(ET, m)
    out = _tc_head(hidden_tc, partials, W, b.reshape(2, 1))
    return out.reshape(2)


# speedup vs baseline: 1.3599x; 1.0651x over previous
import functools

import jax
import jax.numpy as jnp
from jax import lax
from jax.experimental import pallas as pl
from jax.experimental.pallas import tpu as pltpu
from jax.experimental.pallas import tpu_sc as plsc

SEQ = 16384
DIM = 64
VOCAB = 1000000
NUM_CORES = 2
NUM_SUBCORES = 16
NUM_TILES = NUM_CORES * NUM_SUBCORES
PER_SUBCORE = SEQ // NUM_SUBCORES
CHUNK = 128
NCHUNK = PER_SUBCORE // CHUNK
L = 16

BV = 8192
NB_HALF = 62
HALF = NB_HALF * BV
H_PER_TILE = HALF // NUM_SUBCORES

T = 41
TC_NB = T + 1
TC_GRID = NUM_CORES * TC_NB
WIN = 512
CC = WIN // L


def _sc_multiplicity(X):
    mesh = plsc.VectorSubcoreMesh(core_axis_name="c", subcore_axis_name="s")

    @functools.partial(
        pl.kernel,
        mesh=mesh,
        out_type=jax.ShapeDtypeStruct((NUM_CORES, HALF), jnp.float32),
        scratch_types=[
            pltpu.VMEM((NCHUNK, CHUNK), jnp.int32),
            pltpu.VMEM((CHUNK,), jnp.float32),
            pltpu.VMEM((H_PER_TILE,), jnp.float32),
            pltpu.VMEM_SHARED((HALF + L,), jnp.float32),
            pltpu.SemaphoreType.DMA,
        ],
    )
    def sc_kernel(x_hbm, m_hbm, idx_v, ones_v, zeros_v, m_sh, xsem):
        cid = lax.axis_index("c")
        sid = lax.axis_index("s")
        base = sid * PER_SUBCORE
        lo = cid * HALF
        dump = HALF + (sid % L)

        for j in range(NCHUNK):
            pltpu.async_copy(
                x_hbm.at[pl.ds(base + j * CHUNK, CHUNK)], idx_v.at[j], xsem
            )
        for j in range(NCHUNK):
            pltpu.make_async_copy(
                x_hbm.at[pl.ds(0, CHUNK)], idx_v.at[j], xsem
            ).wait()
        for j in range(NCHUNK):
            for k in range(CHUNK // L):
                v = idx_v[j, pl.ds(k * L, L)] - lo
                ok = (v >= 0) & (v < HALF)
                idx_v[j, pl.ds(k * L, L)] = jnp.where(ok, v, dump)

        for k in range(CHUNK // L):
            ones_v[pl.ds(k * L, L)] = jnp.full((L,), 1.0, jnp.float32)

        def zbody(i, _):
            zeros_v[pl.ds(i * L, L)] = jnp.zeros((L,), jnp.float32)
            return 0

        lax.fori_loop(0, H_PER_TILE // L, zbody, 0, unroll=8)

        tslice = pl.ds(sid * H_PER_TILE, H_PER_TILE)
        pltpu.sync_copy(zeros_v, m_sh.at[tslice])
        plsc.subcore_barrier()

        for j in range(NCHUNK):
            pltpu.sync_copy(ones_v, m_sh.at[idx_v.at[j]], add=True)
        plsc.subcore_barrier()

        pltpu.sync_copy(m_sh.at[tslice], m_hbm.at[cid, tslice])

    return sc_kernel(X)


def _sc_stream(ET, m):
    mesh = plsc.VectorSubcoreMesh(core_axis_name="c", subcore_axis_name="s")

    @functools.partial(
        pl.kernel,
        mesh=mesh,
        out_type=jax.ShapeDtypeStruct((NUM_TILES, DIM, L), jnp.float32),
        scratch_types=[
            pltpu.VMEM((4, 8, WIN), jnp.float32),
            pltpu.VMEM((NB_HALF - 1 - T, WIN), jnp.float32),
            pltpu.VMEM((DIM, L), jnp.float32),
            pltpu.SemaphoreType.DMA((4,)),
            pltpu.SemaphoreType.DMA,
        ],
    )
    def sc_kernel(et_hbm, m_hbm, out_hbm, ebuf, mbuf, acc_v, esem, msem):
        cid = lax.axis_index("c")
        sid = lax.axis_index("s")
        wid = cid * NUM_SUBCORES + sid
        nblk = jnp.where(cid == 0, NB_HALF - 1 - T, NB_HALF - 2 - T)
        nchunks = nblk * 8
        MAXBLK = NB_HALF - 1 - T

        def zinit(r, _):
            acc_v[r] = jnp.zeros((L,), jnp.float32)
            return 0

        lax.fori_loop(0, DIM, zinit, 0)

        def col0_of(blk):
            return (cid * NB_HALF + T + blk) * BV + sid * WIN

        for w in range(MAXBLK):
            @pl.when(w < nblk)
            def _():
                pltpu.async_copy(
                    m_hbm.at[cid, pl.ds((T + w) * BV + sid * WIN, WIN)],
                    mbuf.at[w],
                    msem,
                )

        def fire_e(ci, p):
            blk, e8 = ci // 8, ci % 8
            return pltpu.async_copy(
                et_hbm.at[pl.ds(e8 * 8, 8), pl.ds(col0_of(blk), WIN)],
                ebuf.at[p],
                esem.at[p],
            )

        def drain_e(p):
            pltpu.make_async_copy(
                et_hbm.at[pl.ds(0, 8), pl.ds(0, WIN)], ebuf.at[p], esem.at[p]
            ).wait()

        for s in range(4):
            @pl.when(s < nchunks)
            def _():
                fire_e(s, s)

        for w in range(MAXBLK):
            @pl.when(w < nblk)
            def _():
                pltpu.make_async_copy(
                    m_hbm.at[0, pl.ds(0, WIN)], mbuf.at[w], msem
                ).wait()

        def body(ci, _):
            p = lax.rem(ci, 4)
            blk = ci // 8
            drain_e(p)
            e8 = ci % 8

            def inner(cc, accs):
                m16 = mbuf[blk, pl.ds(cc * L, L)]
                out = []
                for r in range(8):
                    x = jnp.maximum(ebuf[p, r, pl.ds(cc * L, L)], 0.0)
                    out.append(accs[r] + x * m16)
                return tuple(out)

            z = jnp.zeros((L,), jnp.float32)
            accs = lax.fori_loop(0, CC, inner, (z,) * 8, unroll=4)
            for r in range(8):
                acc_v[e8 * 8 + r] = acc_v[e8 * 8 + r] + accs[r]

            @pl.when(ci + 4 < nchunks)
            def _():
                fire_e(ci + 4, p)

            return 0

        lax.fori_loop(0, nchunks, body, 0)
        pltpu.sync_copy(acc_v, out_hbm.at[wid])

    return sc_kernel(ET, m)


def _tc_stream(ET, m):
    def tc_kernel(e_ref, m_ref, o_ref, acc_ref):
        i = pl.program_id(0)
        r = i // TC_NB
        j = i % TC_NB
        sel = jnp.where(j < T, j, NB_HALF - 1 - r)
        base = (r * NB_HALF + sel) * BV
        cols = jax.lax.broadcasted_iota(jnp.int32, (1, BV), 1) + base
        e = jnp.where(cols < VOCAB, e_ref[...], 0.0)
        mm = jnp.where(r == 0, m_ref[0:1, :], m_ref[1:2, :])
        contrib = jnp.sum(jnp.maximum(e, 0.0) * mm, axis=1, keepdims=True)

        @pl.when(i == 0)
        def _():
            acc_ref[...] = contrib

        @pl.when(i > 0)
        def _():
            acc_ref[...] = acc_ref[...] + contrib

        @pl.when(i == TC_GRID - 1)
        def _():
            o_ref[...] = acc_ref[...]

    def eidx(i):
        r = i // TC_NB
        j = i % TC_NB
        sel = jnp.where(j < T, j, NB_HALF - 1 - r)
        return (0, r * NB_HALF + sel)

    def midx(i):
        r = i // TC_NB
        j = i % TC_NB
        sel = jnp.where(j < T, j, NB_HALF - 1 - r)
        return (0, sel)

    return pl.pallas_call(
        tc_kernel,
        grid=(TC_GRID,),
        in_specs=[
            pl.BlockSpec((DIM, BV), eidx),
            pl.BlockSpec((NUM_CORES, BV), midx),
        ],
        out_specs=pl.BlockSpec((DIM, 1), lambda i: (0, 0)),
        out_shape=jax.ShapeDtypeStruct((DIM, 1), jnp.float32),
        scratch_shapes=[pltpu.VMEM((DIM, 1), jnp.float32)],
    )(ET, m)


def _tc_head(hidden_tc, partials, W, b2):
    def tc_kernel(h_ref, p_ref, w_ref, b_ref, o_ref):
        ps = jnp.sum(p_ref[...], axis=0)
        ones = jnp.ones((L, 1), jnp.float32)
        h_sc = lax.dot_general(
            ps, ones, (((1,), (0,)), ((), ())),
            preferred_element_type=jnp.float32,
        )
        hidden = h_ref[...] + h_sc
        logits = lax.dot_general(
            w_ref[...], hidden, (((1,), (0,)), ((), ())),
            preferred_element_type=jnp.float32,
        ) + b_ref[...]
        mx = jnp.max(logits, axis=0, keepdims=True)
        ex = jnp.exp(logits - mx)
        o_ref[...] = ex / jnp.sum(ex, axis=0, keepdims=True)

    return pl.pallas_call(
        tc_kernel,
        out_shape=jax.ShapeDtypeStruct((2, 1), jnp.float32),
    )(hidden_tc, partials, W, b2)


def kernel(X, E, W, b):
    X = X.astype(jnp.int32)
    ET = E.T
    m = _sc_multiplicity(X)
    hidden_tc = _tc_stream(ET, m)
    partials = _sc_stream(ET, m)
    out = _tc_head(hidden_tc, partials, W, b.reshape(2, 1))
    return out.reshape(2)


# speedup vs baseline: 1.4852x; 1.0921x over previous
import functools

import jax
import jax.numpy as jnp
from jax import lax
from jax.experimental import pallas as pl
from jax.experimental.pallas import tpu as pltpu
from jax.experimental.pallas import tpu_sc as plsc

SEQ = 16384
DIM = 64
VOCAB = 1000000
NUM_CORES = 2
NUM_SUBCORES = 16
NUM_TILES = NUM_CORES * NUM_SUBCORES
PER_SUBCORE = SEQ // NUM_SUBCORES
CHUNK = 128
NCHUNK = PER_SUBCORE // CHUNK
L = 16

BV = 8192
NB_HALF = 62
HALF = NB_HALF * BV
H_PER_TILE = HALF // NUM_SUBCORES

T = 31
TC_NB = T + 1
TC_GRID = NUM_CORES * TC_NB
WIN = 512
CC = WIN // L


def _sc_multiplicity(X):
    mesh = plsc.VectorSubcoreMesh(core_axis_name="c", subcore_axis_name="s")

    @functools.partial(
        pl.kernel,
        mesh=mesh,
        out_type=jax.ShapeDtypeStruct((NUM_CORES, HALF), jnp.float32),
        scratch_types=[
            pltpu.VMEM((NCHUNK, CHUNK), jnp.int32),
            pltpu.VMEM((CHUNK,), jnp.float32),
            pltpu.VMEM((H_PER_TILE,), jnp.float32),
            pltpu.VMEM_SHARED((HALF + L,), jnp.float32),
            pltpu.SemaphoreType.DMA,
        ],
    )
    def sc_kernel(x_hbm, m_hbm, idx_v, ones_v, zeros_v, m_sh, xsem):
        cid = lax.axis_index("c")
        sid = lax.axis_index("s")
        base = sid * PER_SUBCORE
        lo = cid * HALF
        dump = HALF + (sid % L)

        for j in range(NCHUNK):
            pltpu.async_copy(
                x_hbm.at[pl.ds(base + j * CHUNK, CHUNK)], idx_v.at[j], xsem
            )
        for j in range(NCHUNK):
            pltpu.make_async_copy(
                x_hbm.at[pl.ds(0, CHUNK)], idx_v.at[j], xsem
            ).wait()
        for j in range(NCHUNK):
            for k in range(CHUNK // L):
                v = idx_v[j, pl.ds(k * L, L)] - lo
                ok = (v >= 0) & (v < HALF)
                idx_v[j, pl.ds(k * L, L)] = jnp.where(ok, v, dump)

        for k in range(CHUNK // L):
            ones_v[pl.ds(k * L, L)] = jnp.full((L,), 1.0, jnp.float32)

        def zbody(i, _):
            zeros_v[pl.ds(i * L, L)] = jnp.zeros((L,), jnp.float32)
            return 0

        lax.fori_loop(0, H_PER_TILE // L, zbody, 0, unroll=8)

        tslice = pl.ds(sid * H_PER_TILE, H_PER_TILE)
        pltpu.sync_copy(zeros_v, m_sh.at[tslice])
        plsc.subcore_barrier()

        for j in range(NCHUNK):
            pltpu.sync_copy(ones_v, m_sh.at[idx_v.at[j]], add=True)
        plsc.subcore_barrier()

        pltpu.sync_copy(m_sh.at[tslice], m_hbm.at[cid, tslice])

    return sc_kernel(X)


def _sc_stream(ET, m):
    mesh = plsc.VectorSubcoreMesh(core_axis_name="c", subcore_axis_name="s")

    @functools.partial(
        pl.kernel,
        mesh=mesh,
        out_type=jax.ShapeDtypeStruct((NUM_TILES, DIM, L), jnp.float32),
        scratch_types=[
            pltpu.VMEM((4, 8, WIN), jnp.float32),
            pltpu.VMEM((NB_HALF - 1 - T, WIN), jnp.float32),
            pltpu.VMEM((DIM, L), jnp.float32),
            pltpu.SemaphoreType.DMA((4,)),
            pltpu.SemaphoreType.DMA,
        ],
    )
    def sc_kernel(et_hbm, m_hbm, out_hbm, ebuf, mbuf, acc_v, esem, msem):
        cid = lax.axis_index("c")
        sid = lax.axis_index("s")
        wid = cid * NUM_SUBCORES + sid
        nblk = jnp.where(cid == 0, NB_HALF - 1 - T, NB_HALF - 2 - T)
        nchunks = nblk * 8
        MAXBLK = NB_HALF - 1 - T

        def zinit(r, _):
            acc_v[r] = jnp.zeros((L,), jnp.float32)
            return 0

        lax.fori_loop(0, DIM, zinit, 0)

        def col0_of(blk):
            return (cid * NB_HALF + T + blk) * BV + sid * WIN

        for w in range(MAXBLK):
            @pl.when(w < nblk)
            def _():
                pltpu.async_copy(
                    m_hbm.at[cid, pl.ds((T + w) * BV + sid * WIN, WIN)],
                    mbuf.at[w],
                    msem,
                )

        def fire_e(ci, p):
            blk, e8 = ci // 8, ci % 8
            return pltpu.async_copy(
                et_hbm.at[pl.ds(e8 * 8, 8), pl.ds(col0_of(blk), WIN)],
                ebuf.at[p],
                esem.at[p],
            )

        def drain_e(p):
            pltpu.make_async_copy(
                et_hbm.at[pl.ds(0, 8), pl.ds(0, WIN)], ebuf.at[p], esem.at[p]
            ).wait()

        for s in range(4):
            @pl.when(s < nchunks)
            def _():
                fire_e(s, s)

        for w in range(MAXBLK):
            @pl.when(w < nblk)
            def _():
                pltpu.make_async_copy(
                    m_hbm.at[0, pl.ds(0, WIN)], mbuf.at[w], msem
                ).wait()

        def body(ci, _):
            p = lax.rem(ci, 4)
            blk = ci // 8
            drain_e(p)
            e8 = ci % 8

            def inner(cc, accs):
                m16 = mbuf[blk, pl.ds(cc * L, L)]
                out = []
                for r in range(8):
                    x = jnp.maximum(ebuf[p, r, pl.ds(cc * L, L)], 0.0)
                    out.append(accs[r] + x * m16)
                return tuple(out)

            z = jnp.zeros((L,), jnp.float32)
            accs = lax.fori_loop(0, CC, inner, (z,) * 8, unroll=4)
            for r in range(8):
                acc_v[e8 * 8 + r] = acc_v[e8 * 8 + r] + accs[r]

            @pl.when(ci + 4 < nchunks)
            def _():
                fire_e(ci + 4, p)

            return 0

        lax.fori_loop(0, nchunks, body, 0)
        pltpu.sync_copy(acc_v, out_hbm.at[wid])

    return sc_kernel(ET, m)


def _tc_stream(ET, m):
    def tc_kernel(e_ref, m_ref, o_ref, acc_ref):
        i = pl.program_id(0)
        r = i // TC_NB
        j = i % TC_NB
        sel = jnp.where(j < T, j, NB_HALF - 1 - r)
        base = (r * NB_HALF + sel) * BV
        cols = jax.lax.broadcasted_iota(jnp.int32, (1, BV), 1) + base
        e = jnp.where(cols < VOCAB, e_ref[...], 0.0)
        mm = jnp.where(r == 0, m_ref[0:1, :], m_ref[1:2, :])
        contrib = jnp.sum(jnp.maximum(e, 0.0) * mm, axis=1, keepdims=True)

        @pl.when(i == 0)
        def _():
            acc_ref[...] = contrib

        @pl.when(i > 0)
        def _():
            acc_ref[...] = acc_ref[...] + contrib

        @pl.when(i == TC_GRID - 1)
        def _():
            o_ref[...] = acc_ref[...]

    def eidx(i):
        r = i // TC_NB
        j = i % TC_NB
        sel = jnp.where(j < T, j, NB_HALF - 1 - r)
        return (0, r * NB_HALF + sel)

    def midx(i):
        r = i // TC_NB
        j = i % TC_NB
        sel = jnp.where(j < T, j, NB_HALF - 1 - r)
        return (0, sel)

    return pl.pallas_call(
        tc_kernel,
        grid=(TC_GRID,),
        in_specs=[
            pl.BlockSpec((DIM, BV), eidx),
            pl.BlockSpec((NUM_CORES, BV), midx),
        ],
        out_specs=pl.BlockSpec((DIM, 1), lambda i: (0, 0)),
        out_shape=jax.ShapeDtypeStruct((DIM, 1), jnp.float32),
        scratch_shapes=[pltpu.VMEM((DIM, 1), jnp.float32)],
    )(ET, m)


def _tc_head(hidden_tc, partials, W, b2):
    def tc_kernel(h_ref, p_ref, w_ref, b_ref, o_ref):
        ps = jnp.sum(p_ref[...], axis=0)
        ones = jnp.ones((L, 1), jnp.float32)
        h_sc = lax.dot_general(
            ps, ones, (((1,), (0,)), ((), ())),
            preferred_element_type=jnp.float32,
        )
        hidden = h_ref[...] + h_sc
        logits = lax.dot_general(
            w_ref[...], hidden, (((1,), (0,)), ((), ())),
            preferred_element_type=jnp.float32,
        ) + b_ref[...]
        mx = jnp.max(logits, axis=0, keepdims=True)
        ex = jnp.exp(logits - mx)
        o_ref[...] = ex / jnp.sum(ex, axis=0, keepdims=True)

    return pl.pallas_call(
        tc_kernel,
        out_shape=jax.ShapeDtypeStruct((2, 1), jnp.float32),
    )(hidden_tc, partials, W, b2)


def kernel(X, E, W, b):
    X = X.astype(jnp.int32)
    ET = E.T
    m = _sc_multiplicity(X)
    hidden_tc = _tc_stream(ET, m)
    partials = _sc_stream(ET, m)
    out = _tc_head(hidden_tc, partials, W, b.reshape(2, 1))
    return out.reshape(2)
